# baseline (speedup 1.0000x reference)
import functools

import jax
import jax.numpy as jnp
from jax import lax
from jax.experimental import pallas as pl
from jax.experimental.pallas import tpu as pltpu
from jax.experimental.pallas import tpu_sc as plsc

B = 3_200_000
D = 16
V = 100

_info = plsc.get_sparse_core_info()
NC = _info.num_cores
NS = _info.num_subcores
NW = NC * NS
L = 16

CHUNK = 2000
NCHUNK = B // NW // CHUNK
HALF = NCHUNK // 2
UNROLL = 16

_mesh = plsc.VectorSubcoreMesh(core_axis_name="c", subcore_axis_name="s")


@functools.partial(
    pl.kernel,
    mesh=_mesh,
    compiler_params=pltpu.CompilerParams(use_tc_tiling_on_sc=False,
                                         needs_layout_passes=False),
    out_type=jax.ShapeDtypeStruct((B, D), jnp.float32),
    scratch_types=[
        pltpu.VMEM((V * D,), jnp.float32),
        pltpu.VMEM((CHUNK,), jnp.int32),
        pltpu.VMEM((CHUNK,), jnp.int32),
        pltpu.VMEM((CHUNK, D), jnp.float32),
        pltpu.VMEM((CHUNK, D), jnp.float32),
        pltpu.SemaphoreType.DMA,
        pltpu.SemaphoreType.DMA,
        pltpu.SemaphoreType.DMA,
        pltpu.SemaphoreType.DMA,
    ],
)
def _emb_lookup(idx_hbm, table_hbm, out_hbm, tab_v, idx_v0, idx_v1,
                out_v0, out_v1, sem_i0, sem_i1, sem_o0, sem_o1):
    wid = lax.axis_index("s") * NC + lax.axis_index("c")
    base = wid * NCHUNK

    pltpu.sync_copy(table_hbm, tab_v)

    def idx_slice(c):
        return idx_hbm.at[pl.ds((base + c) * CHUNK, CHUNK)]

    def out_slice(c):
        return out_hbm.at[pl.ds((base + c) * CHUNK, CHUNK), :]

    def compute(idx_v, out_v):
        def cbody(t, carry):
            svec = idx_v[pl.ds(t * L, L)] * D
            rbase = t * L
            for u in range(L):
                out_v[rbase + u, :] = tab_v[pl.ds(svec[u], D)]
            return carry

        lax.fori_loop(0, CHUNK // L, cbody, 0)

    pltpu.async_copy(idx_slice(0), idx_v0, sem_i0)
    pltpu.async_copy(idx_slice(1), idx_v1, sem_i1)

    def body(t, carry):
        a = 2 * t
        bch = a + 1

        pltpu.make_async_copy(idx_slice(a), idx_v0, sem_i0).wait()

        @pl.when(t > 0)
        def _drain_o0():
            pltpu.make_async_copy(out_v0, out_slice(a - 2), sem_o0).wait()

        compute(idx_v0, out_v0)
        pltpu.async_copy(out_v0, out_slice(a), sem_o0)

        @pl.when(t < HALF - 1)
        def _pref_i0():
            pltpu.async_copy(idx_slice(a + 2), idx_v0, sem_i0)

        pltpu.make_async_copy(idx_slice(bch), idx_v1, sem_i1).wait()

        @pl.when(t > 0)
        def _drain_o1():
            pltpu.make_async_copy(out_v1, out_slice(bch - 2), sem_o1).wait()

        compute(idx_v1, out_v1)
        pltpu.async_copy(out_v1, out_slice(bch), sem_o1)

        @pl.when(t < HALF - 1)
        def _pref_i1():
            pltpu.async_copy(idx_slice(bch + 2), idx_v1, sem_i1)

        return carry

    lax.fori_loop(0, HALF, body, 0)

    pltpu.make_async_copy(out_v0, out_slice(NCHUNK - 2), sem_o0).wait()
    pltpu.make_async_copy(out_v1, out_slice(NCHUNK - 1), sem_o1).wait()


def kernel(species, table):
    idx = species.astype(jnp.int32)
    return _emb_lookup(idx, table.reshape(V * D))


# speedup vs baseline: 1.0014x; 1.0014x over previous
import functools

import jax
import jax.numpy as jnp
from jax import lax
from jax.experimental import pallas as pl
from jax.experimental.pallas import tpu as pltpu
from jax.experimental.pallas import tpu_sc as plsc

B = 3_200_000
D = 16
V = 100

_info = plsc.get_sparse_core_info()
NC = _info.num_cores
NS = _info.num_subcores
NW = NC * NS
L = 16

CHUNK = 2000
NCHUNK = B // NW // CHUNK
HALF = NCHUNK // 2
UNROLL = 16

_mesh = plsc.VectorSubcoreMesh(core_axis_name="c", subcore_axis_name="s")


@functools.partial(
    pl.kernel,
    mesh=_mesh,
    compiler_params=pltpu.CompilerParams(use_tc_tiling_on_sc=True,
                                         needs_layout_passes=False),
    out_type=jax.ShapeDtypeStruct((B * D,), jnp.float32),
    scratch_types=[
        pltpu.VMEM((V * D,), jnp.float32),
        pltpu.VMEM((CHUNK,), jnp.int32),
        pltpu.VMEM((CHUNK,), jnp.int32),
        pltpu.VMEM((CHUNK * D,), jnp.float32),
        pltpu.VMEM((CHUNK * D,), jnp.float32),
        pltpu.SemaphoreType.DMA,
        pltpu.SemaphoreType.DMA,
        pltpu.SemaphoreType.DMA,
        pltpu.SemaphoreType.DMA,
    ],
)
def _emb_lookup(idx_hbm, table_hbm, out_hbm, tab_v, idx_v0, idx_v1,
                out_v0, out_v1, sem_i0, sem_i1, sem_o0, sem_o1):
    wid = lax.axis_index("s") * NC + lax.axis_index("c")
    base = wid * NCHUNK

    pltpu.sync_copy(table_hbm, tab_v)

    def idx_slice(c):
        return idx_hbm.at[pl.ds((base + c) * CHUNK, CHUNK)]

    def out_slice(c):
        return out_hbm.at[pl.ds((base + c) * CHUNK * D, CHUNK * D)]

    def compute(idx_v, out_v):
        def cbody(t, carry):
            svec = idx_v[pl.ds(t * L, L)] * D
            rbase = t * L * D
            for u in range(L):
                out_v[pl.ds(rbase + u * D, D)] = tab_v[pl.ds(svec[u], D)]
            return carry

        lax.fori_loop(0, CHUNK // L, cbody, 0)

    pltpu.async_copy(idx_slice(0), idx_v0, sem_i0)
    pltpu.async_copy(idx_slice(1), idx_v1, sem_i1)

    def body(t, carry):
        a = 2 * t
        bch = a + 1

        pltpu.make_async_copy(idx_slice(a), idx_v0, sem_i0).wait()

        @pl.when(t > 0)
        def _drain_o0():
            pltpu.make_async_copy(out_v0, out_slice(a - 2), sem_o0).wait()

        compute(idx_v0, out_v0)
        pltpu.async_copy(out_v0, out_slice(a), sem_o0)

        @pl.when(t < HALF - 1)
        def _pref_i0():
            pltpu.async_copy(idx_slice(a + 2), idx_v0, sem_i0)

        pltpu.make_async_copy(idx_slice(bch), idx_v1, sem_i1).wait()

        @pl.when(t > 0)
        def _drain_o1():
            pltpu.make_async_copy(out_v1, out_slice(bch - 2), sem_o1).wait()

        compute(idx_v1, out_v1)
        pltpu.async_copy(out_v1, out_slice(bch), sem_o1)

        @pl.when(t < HALF - 1)
        def _pref_i1():
            pltpu.async_copy(idx_slice(bch + 2), idx_v1, sem_i1)

        return carry

    lax.fori_loop(0, HALF, body, 0)

    pltpu.make_async_copy(out_v0, out_slice(NCHUNK - 2), sem_o0).wait()
    pltpu.make_async_copy(out_v1, out_slice(NCHUNK - 1), sem_o1).wait()


def kernel(species, table):
    idx = species.astype(jnp.int32)
    out = _emb_lookup(idx, table.reshape(V * D))
    return out.reshape(B, D)


# speedup vs baseline: 2.1826x; 2.1795x over previous
import functools

import jax
import jax.numpy as jnp
from jax import lax
from jax.experimental import pallas as pl
from jax.experimental.pallas import tpu as pltpu
from jax.experimental.pallas import tpu_sc as plsc

B = 3_200_000
D = 16
V = 100

_info = plsc.get_sparse_core_info()
NC = _info.num_cores
NS = _info.num_subcores
NW = NC * NS
L = 16

NT = B // 128
G = 8
CHUNK = G * 128
NCH = NT // G
CH_LO = NCH // NW
NW_HI = NCH - CH_LO * NW

_mesh = plsc.VectorSubcoreMesh(core_axis_name="c", subcore_axis_name="s")


@functools.partial(
    pl.kernel,
    mesh=_mesh,
    compiler_params=pltpu.CompilerParams(use_tc_tiling_on_sc=True,
                                         needs_layout_passes=False),
    out_type=jax.ShapeDtypeStruct((2, NT, 8, 128), jnp.float32),
    scratch_types=[
        pltpu.VMEM((V * 256,), jnp.float32),
        pltpu.VMEM((CHUNK,), jnp.int32),
        pltpu.VMEM((2, G, 8, 128), jnp.float32),
        pltpu.SemaphoreType.DMA,
    ],
)
def _emb_lookup(idx_hbm, table_hbm, out_hbm, tab_v, idx_v, out_v, sem):
    wid = lax.axis_index("s") * NC + lax.axis_index("c")
    start = wid * CH_LO + jnp.minimum(wid, NW_HI)
    nch = CH_LO + jnp.where(wid < NW_HI, 1, 0)

    pltpu.sync_copy(table_hbm, tab_v)

    iota = lax.iota(jnp.int32, L)
    cvecs = [iota + c * L for c in range(D)]

    def body(t, carry):
        pltpu.sync_copy(idx_hbm.at[pl.ds((start + t) * CHUNK, CHUNK)], idx_v)

        def gbody(g, gcarry):
            for sub in range(8):
                svec = idx_v[pl.ds(g * 128 + sub * L, L)] * 256
                for c in range(D):
                    val = plsc.load_gather(tab_v, [svec + cvecs[c]])
                    out_v[c // 8, g, c % 8, pl.ds(sub * L, L)] = val
            return gcarry

        lax.fori_loop(0, G, gbody, 0)

        for h in range(2):
            pltpu.sync_copy(out_v.at[h],
                            out_hbm.at[h, pl.ds((start + t) * G, G), :, :])
        return carry

    lax.fori_loop(0, nch, body, 0)


def kernel(species, table):
    idx = species.astype(jnp.int32)
    tab_rep = jnp.repeat(table.reshape(V * D), L, total_repeat_length=V * 256)
    out4 = _emb_lookup(idx, tab_rep)
    return out4.transpose((1, 3, 0, 2)).reshape(B, D)


# speedup vs baseline: 3.4002x; 1.5579x over previous
import functools

import jax
import jax.numpy as jnp
from jax import lax
from jax.experimental import pallas as pl
from jax.experimental.pallas import tpu as pltpu
from jax.experimental.pallas import tpu_sc as plsc

B = 3_200_000
D = 16
V = 100

_info = plsc.get_sparse_core_info()
NC = _info.num_cores
NS = _info.num_subcores
NW = NC * NS
L = 16

NT = B // 128
G = 8
CHUNK = G * 128
NCH = NT // G
CH_LO = NCH // NW
NW_HI = NCH - CH_LO * NW

_mesh = plsc.VectorSubcoreMesh(core_axis_name="c", subcore_axis_name="s")


@functools.partial(
    pl.kernel,
    mesh=_mesh,
    compiler_params=pltpu.CompilerParams(use_tc_tiling_on_sc=True,
                                         needs_layout_passes=False),
    out_type=jax.ShapeDtypeStruct((2, NT, 8, 128), jnp.float32),
    scratch_types=[
        pltpu.VMEM((V * 256,), jnp.float32),
        pltpu.VMEM((CHUNK,), jnp.int32),
        pltpu.VMEM((2, G, 8, 128), jnp.float32),
        pltpu.SemaphoreType.DMA,
    ],
)
def _emb_lookup(idx_hbm, table_hbm, out_hbm, tab_v, idx_v, out_v, sem):
    wid = lax.axis_index("s") * NC + lax.axis_index("c")
    start = wid * CH_LO + jnp.minimum(wid, NW_HI)
    nch = CH_LO + jnp.where(wid < NW_HI, 1, 0)

    pltpu.sync_copy(table_hbm, tab_v)

    iota = lax.iota(jnp.int32, L)
    cvecs = [iota + c * L for c in range(D)]

    def body(t, carry):
        pltpu.sync_copy(idx_hbm.at[pl.ds((start + t) * CHUNK, CHUNK)], idx_v)

        def gbody(g, gcarry):
            for sub in range(8):
                svec = idx_v[pl.ds(g * 128 + sub * L, L)] * 256
                for c in range(D):
                    val = plsc.load_gather(tab_v, [svec + cvecs[c]])
                    out_v[c // 8, g, c % 8, pl.ds(sub * L, L)] = val
            return gcarry

        lax.fori_loop(0, G, gbody, 0)

        for h in range(2):
            pltpu.sync_copy(out_v.at[h],
                            out_hbm.at[h, pl.ds((start + t) * G, G), :, :])
        return carry

    lax.fori_loop(0, nch, body, 0)


def kernel(species, table):
    idx = species.astype(jnp.int32)
    tab_rep = jnp.broadcast_to(table.reshape(V * D, 1), (V * D, L)).reshape(-1)
    out4 = _emb_lookup(idx, tab_rep)
    return out4.transpose((1, 3, 0, 2)).reshape(B, D)


# speedup vs baseline: 4.4855x; 1.3192x over previous
import functools

import jax
import jax.numpy as jnp
from jax import lax
from jax.experimental import pallas as pl
from jax.experimental.pallas import tpu as pltpu
from jax.experimental.pallas import tpu_sc as plsc

B = 3_200_000
D = 16
V = 100

_info = plsc.get_sparse_core_info()
NC = _info.num_cores
NS = _info.num_subcores
NW = NC * NS
L = 16

NT = B // 128
G = 10
CHUNK = G * 128
NCH = NT // G
CH_LO = NCH // NW
NW_HI = NCH - CH_LO * NW

_mesh = plsc.VectorSubcoreMesh(core_axis_name="c", subcore_axis_name="s")


@functools.partial(
    pl.kernel,
    mesh=_mesh,
    compiler_params=pltpu.CompilerParams(use_tc_tiling_on_sc=True,
                                         needs_layout_passes=False),
    out_type=jax.ShapeDtypeStruct((2, NT, 8, 128), jnp.float32),
    scratch_types=[
        pltpu.VMEM((V * 256,), jnp.float32),
        pltpu.VMEM((2, CHUNK), jnp.int32),
        pltpu.VMEM((2, 2, G, 8, 128), jnp.float32),
        pltpu.SemaphoreType.DMA((2,)),
        pltpu.SemaphoreType.DMA((2,)),
    ],
)
def _emb_lookup(idx_hbm, table_hbm, out_hbm, tab_v, idx_v, out_v,
                isem, osem):
    wid = lax.axis_index("s") * NC + lax.axis_index("c")
    start = wid * CH_LO + jnp.minimum(wid, NW_HI)
    nch = CH_LO + jnp.where(wid < NW_HI, 1, 0)

    pltpu.sync_copy(table_hbm, tab_v)

    iota = lax.iota(jnp.int32, L)
    cvecs = [iota + c * L for c in range(D)]

    def idx_slice(t):
        return idx_hbm.at[pl.ds((start + t) * CHUNK, CHUNK)]

    def out_slice(t, h):
        return out_hbm.at[h, pl.ds((start + t) * G, G), :, :]

    def compute(p):
        def gbody(g, gcarry):
            for sub in range(8):
                svec = idx_v[p, pl.ds(g * 128 + sub * L, L)] * 256
                for c in range(D):
                    val = plsc.load_gather(tab_v, [svec + cvecs[c]])
                    out_v[p, c // 8, g, c % 8, pl.ds(sub * L, L)] = val
            return gcarry

        lax.fori_loop(0, G, gbody, 0)

    pltpu.async_copy(idx_slice(0), idx_v.at[0], isem.at[0])

    @pl.when(nch > 1)
    def _prime2():
        pltpu.async_copy(idx_slice(1), idx_v.at[1], isem.at[1])

    def body(t, carry):
        p = lax.rem(t, 2)
        pltpu.make_async_copy(idx_slice(t), idx_v.at[p], isem.at[p]).wait()

        @pl.when(t > 1)
        def _drain():
            for h in range(2):
                pltpu.make_async_copy(out_v.at[p, h], out_slice(t - 2, h),
                                      osem.at[p]).wait()

        compute(p)
        for h in range(2):
            pltpu.async_copy(out_v.at[p, h], out_slice(t, h), osem.at[p])

        @pl.when(t + 2 < nch)
        def _prefetch():
            pltpu.async_copy(idx_slice(t + 2), idx_v.at[p], isem.at[p])

        return carry

    lax.fori_loop(0, nch, body, 0)

    def drain(t):
        @pl.when(t >= 0)
        def _():
            p = lax.rem(t, 2)
            for h in range(2):
                pltpu.make_async_copy(out_v.at[p, h], out_slice(t, h),
                                      osem.at[p]).wait()

    drain(nch - 2)
    drain(nch - 1)


def kernel(species, table):
    idx = species.astype(jnp.int32)
    tab_rep = jnp.broadcast_to(table.reshape(V * D, 1), (V * D, L)).reshape(-1)
    out4 = _emb_lookup(idx, tab_rep)
    return out4.transpose((1, 3, 0, 2)).reshape(B, D)


# speedup vs baseline: 10.3480x; 2.3070x over previous
import functools

import jax
import jax.numpy as jnp
from jax import lax
from jax.experimental import pallas as pl
from jax.experimental.pallas import tpu as pltpu
from jax.experimental.pallas import tpu_sc as plsc

B = 3_200_000
D = 16
V = 100

_info = plsc.get_sparse_core_info()
NC = _info.num_cores
NS = _info.num_subcores
NW = NC * NS
L = 16

NT = B // 128
G = 10
CHUNK = G * 128
NCH = NT // G
CH_LO = NCH // NW
NW_HI = NCH - CH_LO * NW

_mesh = plsc.VectorSubcoreMesh(core_axis_name="c", subcore_axis_name="s")


@functools.partial(
    pl.kernel,
    mesh=_mesh,
    compiler_params=pltpu.CompilerParams(use_tc_tiling_on_sc=True,
                                         needs_layout_passes=False),
    out_type=jax.ShapeDtypeStruct((2, NT, 8, 128), jnp.float32),
    scratch_types=[
        pltpu.VMEM((V * 256,), jnp.float32),
        pltpu.VMEM((2, CHUNK), jnp.int32),
        pltpu.VMEM((2, 2, G, 8, 128), jnp.float32),
        pltpu.SemaphoreType.DMA((2,)),
        pltpu.SemaphoreType.DMA((2,)),
    ],
)
def _emb_lookup(idx_hbm, table_hbm, out_hbm, tab_v, idx_v, out_v,
                isem, osem):
    wid = lax.axis_index("s") * NC + lax.axis_index("c")
    start = wid * CH_LO + jnp.minimum(wid, NW_HI)
    nch = CH_LO + jnp.where(wid < NW_HI, 1, 0)

    pltpu.sync_copy(table_hbm, tab_v)

    iota = lax.iota(jnp.int32, L)
    cvecs = [iota + c * L for c in range(D)]

    def idx_slice(t):
        return idx_hbm.at[pl.ds((start + t) * CHUNK, CHUNK)]

    def out_slice(t, h):
        return out_hbm.at[h, pl.ds((start + t) * G, G), :, :]

    def compute(p):
        def gbody(g, gcarry):
            for sub in range(8):
                svec = idx_v[p, pl.ds(g * 128 + sub * L, L)] * 256
                addrs = [svec + cvecs[c] for c in range(D)]
                vals = [plsc.load_gather(tab_v, [addrs[c]]) for c in range(D)]
                for c in range(D):
                    out_v[p, c // 8, g, c % 8, pl.ds(sub * L, L)] = vals[c]
            return gcarry

        lax.fori_loop(0, G, gbody, 0)

    pltpu.async_copy(idx_slice(0), idx_v.at[0], isem.at[0])

    @pl.when(nch > 1)
    def _prime2():
        pltpu.async_copy(idx_slice(1), idx_v.at[1], isem.at[1])

    def body(t, carry):
        p = lax.rem(t, 2)
        pltpu.make_async_copy(idx_slice(t), idx_v.at[p], isem.at[p]).wait()

        @pl.when(t > 1)
        def _drain():
            for h in range(2):
                pltpu.make_async_copy(out_v.at[p, h], out_slice(t - 2, h),
                                      osem.at[p]).wait()

        compute(p)
        for h in range(2):
            pltpu.async_copy(out_v.at[p, h], out_slice(t, h), osem.at[p])

        @pl.when(t + 2 < nch)
        def _prefetch():
            pltpu.async_copy(idx_slice(t + 2), idx_v.at[p], isem.at[p])

        return carry

    lax.fori_loop(0, nch, body, 0)

    def drain(t):
        @pl.when(t >= 0)
        def _():
            p = lax.rem(t, 2)
            for h in range(2):
                pltpu.make_async_copy(out_v.at[p, h], out_slice(t, h),
                                      osem.at[p]).wait()

    drain(nch - 2)
    drain(nch - 1)


def kernel(species, table):
    idx = species.astype(jnp.int32)
    tab_rep = jnp.broadcast_to(table.reshape(V * D, 1), (V * D, L)).reshape(-1)
    out4 = _emb_lookup(idx, tab_rep)
    return out4.transpose((1, 3, 0, 2)).reshape(B, D)


# speedup vs baseline: 10.3607x; 1.0012x over previous
import functools

import jax
import jax.numpy as jnp
from jax import lax
from jax.experimental import pallas as pl
from jax.experimental.pallas import tpu as pltpu
from jax.experimental.pallas import tpu_sc as plsc

B = 3_200_000
D = 16
V = 100

_info = plsc.get_sparse_core_info()
NC = _info.num_cores
NS = _info.num_subcores
NW = NC * NS
L = 16

NT = B // 128
G = 5
NB = 4
CHUNK = G * 128
NCH = NT // G
CH_LO = NCH // NW
NW_HI = NCH - CH_LO * NW

_mesh = plsc.VectorSubcoreMesh(core_axis_name="c", subcore_axis_name="s")


@functools.partial(
    pl.kernel,
    mesh=_mesh,
    compiler_params=pltpu.CompilerParams(use_tc_tiling_on_sc=True,
                                         needs_layout_passes=False),
    out_type=jax.ShapeDtypeStruct((2, NT, 8, 128), jnp.float32),
    scratch_types=[
        pltpu.VMEM((V * 256,), jnp.float32),
        pltpu.VMEM((NB, CHUNK), jnp.int32),
        pltpu.VMEM((NB, 2, G, 8, 128), jnp.float32),
        pltpu.SemaphoreType.DMA((NB,)),
        pltpu.SemaphoreType.DMA((NB,)),
    ],
)
def _emb_lookup(idx_hbm, table_hbm, out_hbm, tab_v, idx_v, out_v,
                isem, osem):
    wid = lax.axis_index("s") * NC + lax.axis_index("c")
    start = wid * CH_LO + jnp.minimum(wid, NW_HI)
    nch = CH_LO + jnp.where(wid < NW_HI, 1, 0)

    pltpu.sync_copy(table_hbm, tab_v)

    iota = lax.iota(jnp.int32, L)
    cvecs = [iota + c * L for c in range(D)]

    def idx_slice(t):
        return idx_hbm.at[pl.ds((start + t) * CHUNK, CHUNK)]

    def out_slice(t, h):
        return out_hbm.at[h, pl.ds((start + t) * G, G), :, :]

    def compute(p):
        def gbody(g, gcarry):
            for sub in range(8):
                svec = idx_v[p, pl.ds(g * 128 + sub * L, L)] * 256
                addrs = [svec + cvecs[c] for c in range(D)]
                vals = [plsc.load_gather(tab_v, [addrs[c]]) for c in range(D)]
                for c in range(D):
                    out_v[p, c // 8, g, c % 8, pl.ds(sub * L, L)] = vals[c]
            return gcarry

        lax.fori_loop(0, G, gbody, 0)

    for k in range(NB):
        @pl.when(nch > k)
        def _prime(k=k):
            pltpu.async_copy(idx_slice(k), idx_v.at[k], isem.at[k])

    def body(t, carry):
        p = lax.rem(t, NB)
        pltpu.make_async_copy(idx_slice(t), idx_v.at[p], isem.at[p]).wait()

        @pl.when(t >= NB)
        def _drain():
            for h in range(2):
                pltpu.make_async_copy(out_v.at[p, h], out_slice(t - NB, h),
                                      osem.at[p]).wait()

        compute(p)
        for h in range(2):
            pltpu.async_copy(out_v.at[p, h], out_slice(t, h), osem.at[p])

        @pl.when(t + NB < nch)
        def _prefetch():
            pltpu.async_copy(idx_slice(t + NB), idx_v.at[p], isem.at[p])

        return carry

    lax.fori_loop(0, nch, body, 0)

    for k in range(NB):
        t = nch - NB + k

        @pl.when(t >= 0)
        def _draink(t=t):
            p = lax.rem(t, NB)
            for h in range(2):
                pltpu.make_async_copy(out_v.at[p, h], out_slice(t, h),
                                      osem.at[p]).wait()


def kernel(species, table):
    idx = species.astype(jnp.int32)
    tab_rep = jnp.broadcast_to(table.reshape(V * D, 1), (V * D, L)).reshape(-1)
    out4 = _emb_lookup(idx, tab_rep)
    return out4.transpose((1, 3, 0, 2)).reshape(B, D)


# speedup vs baseline: 12.8088x; 1.2363x over previous
import functools

import jax
import jax.numpy as jnp
from jax import lax
from jax.experimental import pallas as pl
from jax.experimental.pallas import tpu as pltpu
from jax.experimental.pallas import tpu_sc as plsc

B = 3_200_000
D = 16
V = 100

_info = plsc.get_sparse_core_info()
NC = _info.num_cores
NS = _info.num_subcores
NW = NC * NS
L = 16

TABR = V * D * L
NT = B // 128
G = 5
NB = 4
CHUNK = G * 128
NCH = NT // G
CH_LO = NCH // NW
NW_HI = NCH - CH_LO * NW

_mesh = plsc.VectorSubcoreMesh(core_axis_name="c", subcore_axis_name="s")


@functools.partial(
    pl.kernel,
    mesh=_mesh,
    compiler_params=pltpu.CompilerParams(use_tc_tiling_on_sc=True,
                                         needs_layout_passes=False),
    out_type=jax.ShapeDtypeStruct((2, NT, 8, 128), jnp.float32),
    scratch_types=[
        pltpu.VMEM((TABR + (D - 1) * L,), jnp.float32),
        pltpu.VMEM((NB, CHUNK), jnp.int32),
        pltpu.VMEM((NB, 2, G, 8, 128), jnp.float32),
        pltpu.SemaphoreType.DMA((NB,)),
        pltpu.SemaphoreType.DMA((NB,)),
    ],
)
def _emb_lookup(idx_hbm, table_hbm, out_hbm, tab_v, idx_v, out_v,
                isem, osem):
    wid = lax.axis_index("s") * NC + lax.axis_index("c")
    start = wid * CH_LO + jnp.minimum(wid, NW_HI)
    nch = CH_LO + jnp.where(wid < NW_HI, 1, 0)

    pltpu.sync_copy(table_hbm, tab_v.at[pl.ds(0, TABR)])

    iota = lax.iota(jnp.int32, L)
    tslice = [tab_v.at[pl.ds(c * L, TABR)] for c in range(D)]

    def idx_slice(t):
        return idx_hbm.at[pl.ds((start + t) * CHUNK, CHUNK)]

    def out_slice(t, h):
        return out_hbm.at[h, pl.ds((start + t) * G, G), :, :]

    def compute(p):
        def gbody(g, gcarry):
            avecs = [
                idx_v[p, pl.ds(g * 128 + sub * L, L)] * 256 + iota
                for sub in range(8)
            ]
            for sub in range(8):
                vals = [plsc.load_gather(tslice[c], [avecs[sub]])
                        for c in range(D)]
                for c in range(D):
                    out_v[p, c // 8, g, c % 8, pl.ds(sub * L, L)] = vals[c]
            return gcarry

        lax.fori_loop(0, G, gbody, 0)

    for k in range(NB):
        @pl.when(nch > k)
        def _prime(k=k):
            pltpu.async_copy(idx_slice(k), idx_v.at[k], isem.at[k])

    def body(t, carry):
        p = lax.rem(t, NB)
        pltpu.make_async_copy(idx_slice(t), idx_v.at[p], isem.at[p]).wait()

        @pl.when(t >= NB)
        def _drain():
            for h in range(2):
                pltpu.make_async_copy(out_v.at[p, h], out_slice(t - NB, h),
                                      osem.at[p]).wait()

        compute(p)
        for h in range(2):
            pltpu.async_copy(out_v.at[p, h], out_slice(t, h), osem.at[p])

        @pl.when(t + NB < nch)
        def _prefetch():
            pltpu.async_copy(idx_slice(t + NB), idx_v.at[p], isem.at[p])

        return carry

    lax.fori_loop(0, nch, body, 0)

    for k in range(NB):
        t = nch - NB + k

        @pl.when(t >= 0)
        def _draink(t=t):
            p = lax.rem(t, NB)
            for h in range(2):
                pltpu.make_async_copy(out_v.at[p, h], out_slice(t, h),
                                      osem.at[p]).wait()


def kernel(species, table):
    idx = species.astype(jnp.int32)
    tab_rep = jnp.broadcast_to(table.reshape(V * D, 1), (V * D, L)).reshape(-1)
    out4 = _emb_lookup(idx, tab_rep)
    return out4.transpose((1, 3, 0, 2)).reshape(B, D)
